# Initial kernel scaffold; baseline (speedup 1.0000x reference)
import jax
import jax.numpy as jnp
from jax.experimental import pallas as pl


def kernel(node_attr, edge_index, edge_attr, W1e, b1e, W2e, b2e, W1n, b1n, W2n, b2n):
    raise NotImplementedError("write your pallas kernel here")


# speedup vs baseline: 2.4444x; 2.4444x over previous
import functools

import jax
import jax.numpy as jnp
from jax import lax
from jax.experimental import pallas as pl
from jax.experimental.pallas import tpu as pltpu
from jax.experimental.pallas import tpu_sc as plsc

N = 10000
E = 320000
D = 128

NC = 2
NS = 16
NW = NC * NS
EPW = E // NW
CHUNK = 80
NCHUNK = EPW // CHUNK
RPS = 624
RPS_LAST = N - RPS * (NS - 1)
LANES = 16

_mesh = plsc.VectorSubcoreMesh(core_axis_name="c", subcore_axis_name="s")


def _pre_body(na_ref, w_ref, ps_ref, pr_ref):
    p = jnp.dot(na_ref[...], w_ref[...], preferred_element_type=jnp.float32)
    ps_ref[...] = p[:, :D]
    pr_ref[...] = p[:, D:]


def _precompute(node_attr, w_sr):
    return pl.pallas_call(
        _pre_body,
        out_shape=(
            jax.ShapeDtypeStruct((N, D), jnp.float32),
            jax.ShapeDtypeStruct((N, D), jnp.float32),
        ),
    )(node_attr, w_sr)


@functools.partial(
    pl.kernel,
    out_type=jax.ShapeDtypeStruct((E, D), jnp.float32),
    mesh=_mesh,
    scratch_types=[
        pltpu.VMEM((CHUNK,), jnp.int32),
        pltpu.VMEM((CHUNK,), jnp.int32),
        pltpu.VMEM((CHUNK, D), jnp.float32),
        pltpu.VMEM((CHUNK, D), jnp.float32),
        pltpu.VMEM((CHUNK, D), jnp.float32),
        pltpu.SemaphoreType.DMA,
        pltpu.SemaphoreType.DMA,
    ],
)
def _gather_add(ps_hbm, pr_hbm, s_hbm, r_hbm, out_hbm,
                sidx, ridx, srows, rrows, orows, sem_s, sem_r):
    wid = lax.axis_index("s") * NC + lax.axis_index("c")
    base = wid * EPW

    def chunk(c, carry):
        off = base + c * CHUNK
        pltpu.sync_copy(s_hbm.at[pl.ds(off, CHUNK)], sidx)
        pltpu.sync_copy(r_hbm.at[pl.ds(off, CHUNK)], ridx)
        cp_s = pltpu.async_copy(ps_hbm.at[sidx], srows, sem_s)
        cp_r = pltpu.async_copy(pr_hbm.at[ridx], rrows, sem_r)
        cp_s.wait()
        cp_r.wait()

        def row(i, carry2):
            for j in range(D // LANES):
                sl = pl.ds(j * LANES, LANES)
                orows[i, sl] = srows[i, sl] + rrows[i, sl]
            return carry2

        lax.fori_loop(0, CHUNK, row, 0)
        pltpu.sync_copy(orows, out_hbm.at[pl.ds(off, CHUNK), :])
        return carry

    lax.fori_loop(0, NCHUNK, chunk, 0)


BE = 2560


def _edge_body(g_ref, ea_ref, w1_ref, b1_ref, w2_ref, b2_ref,
               eno_ref, eout_ref, cs_ref):
    i = pl.program_id(0)
    ea = ea_ref[...]
    h = jnp.maximum(
        g_ref[...]
        + jnp.dot(ea, w1_ref[...], preferred_element_type=jnp.float32)
        + b1_ref[...],
        0.0,
    )
    eno = jnp.dot(h, w2_ref[...], preferred_element_type=jnp.float32) + b2_ref[...]
    eno_ref[...] = eno
    eout_ref[...] = eno + ea

    @pl.when(i == 0)
    def _():
        cs_ref[...] = jnp.zeros_like(cs_ref)

    cs_ref[...] += jnp.sum(eno, axis=0, keepdims=True)


def _edge_mlp(g, edge_attr, w1, b1, w2, b2):
    grid = (E // BE,)
    blk = lambda i: (i, 0)
    fix = lambda i: (0, 0)
    return pl.pallas_call(
        _edge_body,
        grid=grid,
        in_specs=[
            pl.BlockSpec((BE, D), blk),
            pl.BlockSpec((BE, D), blk),
            pl.BlockSpec((D, D), fix),
            pl.BlockSpec((1, D), fix),
            pl.BlockSpec((D, D), fix),
            pl.BlockSpec((1, D), fix),
        ],
        out_specs=(
            pl.BlockSpec((BE, D), blk),
            pl.BlockSpec((BE, D), blk),
            pl.BlockSpec((1, D), fix),
        ),
        out_shape=(
            jax.ShapeDtypeStruct((E, D), jnp.float32),
            jax.ShapeDtypeStruct((E, D), jnp.float32),
            jax.ShapeDtypeStruct((1, D), jnp.float32),
        ),
    )(g, edge_attr, w1, b1, w2, b2)


HALF = N // NC
DUST = HALF + 0
AGG_ROWS = HALF + 8 * NS
EPS = E // NS
NCHUNK_S = EPS // CHUNK
ORPS = 312
ORPS_LAST = HALF - ORPS * (NS - 1)


@functools.partial(
    pl.kernel,
    out_type=jax.ShapeDtypeStruct((N, D), jnp.float32),
    mesh=_mesh,
    scratch_types=[
        pltpu.VMEM((CHUNK,), jnp.int32),
        pltpu.VMEM((CHUNK,), jnp.int32),
        pltpu.VMEM((CHUNK, D), jnp.float32),
        pltpu.VMEM((ORPS_LAST, D), jnp.float32),
        pltpu.VMEM_SHARED((AGG_ROWS, D), jnp.float32),
    ],
)
def _scatter_add(eno_hbm, r_hbm, out_hbm, idxb, idxl, rows, big, agg_sh):
    cid = lax.axis_index("c")
    sid = lax.axis_index("s")

    def zrow(i, carry):
        for j in range(D // LANES):
            big[i, pl.ds(j * LANES, LANES)] = jnp.zeros((LANES,), jnp.float32)
        return carry

    lax.fori_loop(0, ORPS_LAST, zrow, 0)

    @pl.when(sid < NS - 1)
    def _():
        pltpu.sync_copy(big.at[pl.ds(0, ORPS), :],
                        agg_sh.at[pl.ds(sid * ORPS, ORPS), :])

    @pl.when(sid == NS - 1)
    def _():
        pltpu.sync_copy(big, agg_sh.at[pl.ds(ORPS * (NS - 1), ORPS_LAST), :])

    pltpu.sync_copy(big.at[pl.ds(0, 8), :],
                    agg_sh.at[pl.ds(DUST + 8 * sid, 8), :])
    plsc.subcore_barrier()

    base = sid * EPS
    lo = cid * HALF
    dust_row = DUST + 8 * sid

    def chunk(c, carry):
        off = base + c * CHUNK
        pltpu.sync_copy(r_hbm.at[pl.ds(off, CHUNK)], idxb)
        pltpu.sync_copy(eno_hbm.at[pl.ds(off, CHUNK), :], rows)
        for j in range(CHUNK // LANES):
            sl = pl.ds(j * LANES, LANES)
            v = idxb[sl] - lo
            ok = (v >= 0) & (v < HALF)
            idxl[sl] = jnp.where(ok, v, dust_row)
        pltpu.sync_copy(rows, agg_sh.at[idxl], add=True)
        return carry

    lax.fori_loop(0, NCHUNK_S, chunk, 0)
    plsc.subcore_barrier()

    @pl.when(sid < NS - 1)
    def _():
        pltpu.sync_copy(agg_sh.at[pl.ds(sid * ORPS, ORPS), :],
                        big.at[pl.ds(0, ORPS), :])
        pltpu.sync_copy(big.at[pl.ds(0, ORPS), :],
                        out_hbm.at[pl.ds(cid * HALF + sid * ORPS, ORPS), :])

    @pl.when(sid == NS - 1)
    def _():
        pltpu.sync_copy(agg_sh.at[pl.ds(ORPS * (NS - 1), ORPS_LAST), :], big)
        pltpu.sync_copy(
            big, out_hbm.at[pl.ds(cid * HALF + ORPS * (NS - 1), ORPS_LAST), :])


def _node_body(na_ref, agg_ref, cs_ref, w1a_ref, w1b_ref, b1_ref,
               w2_ref, b2_ref, out_ref):
    na = na_ref[...]
    aggc = agg_ref[...] - cs_ref[...] * (1.0 / N)
    h = jnp.maximum(
        jnp.dot(na, w1a_ref[...], preferred_element_type=jnp.float32)
        + jnp.dot(aggc, w1b_ref[...], preferred_element_type=jnp.float32)
        + b1_ref[...],
        0.0,
    )
    x = jnp.dot(h, w2_ref[...], preferred_element_type=jnp.float32) + b2_ref[...] + na
    out_ref[...] = x - jnp.mean(x, axis=0, keepdims=True)


def _node_mlp(node_attr, agg, cs, w1a, w1b, b1, w2, b2):
    return pl.pallas_call(
        _node_body,
        out_shape=jax.ShapeDtypeStruct((N, D), jnp.float32),
    )(node_attr, agg, cs, w1a, w1b, b1, w2, b2)


def kernel(node_attr, edge_index, edge_attr, W1e, b1e, W2e, b2e,
           W1n, b1n, W2n, b2n):
    senders = edge_index[0]
    receivers = edge_index[1]

    w_sr = jnp.concatenate([W1e[:D], W1e[D:2 * D]], axis=1)
    ps, pr = _precompute(node_attr, w_sr)
    g = _gather_add(ps, pr, senders, receivers)
    eno, edge_out, cs = _edge_mlp(
        g, edge_attr, W1e[2 * D:], b1e.reshape(1, D), W2e, b2e.reshape(1, D)
    )
    agg = _scatter_add(eno, receivers)
    x = _node_mlp(
        node_attr, agg, cs,
        W1n[:D], W1n[D:], b1n.reshape(1, D), W2n, b2n.reshape(1, D),
    )
    return (x, edge_index, edge_out)


# speedup vs baseline: 3.0148x; 1.2333x over previous
import functools

import jax
import jax.numpy as jnp
from jax import lax
from jax.experimental import pallas as pl
from jax.experimental.pallas import tpu as pltpu
from jax.experimental.pallas import tpu_sc as plsc

N = 10000
E = 320000
D = 128

NC = 2
NS = 16
NW = NC * NS
LANES = 16
CHUNK = 80

EPW = E // NW
NCHUNK = EPW // CHUNK

_mesh = plsc.VectorSubcoreMesh(core_axis_name="c", subcore_axis_name="s")


def _pre_body(na_ref, w_ref, ps_ref, pr_ref):
    p = jnp.dot(na_ref[...], w_ref[...], preferred_element_type=jnp.float32)
    ps_ref[...] = p[:, :D]
    pr_ref[...] = p[:, D:]


def _precompute(node_attr, w_sr):
    return pl.pallas_call(
        _pre_body,
        out_shape=(
            jax.ShapeDtypeStruct((N, D), jnp.float32),
            jax.ShapeDtypeStruct((N, D), jnp.float32),
        ),
    )(node_attr, w_sr)


@functools.partial(
    pl.kernel,
    out_type=jax.ShapeDtypeStruct((E, D), jnp.float32),
    mesh=_mesh,
    scratch_types=[
        pltpu.VMEM((CHUNK,), jnp.int32),
        pltpu.VMEM((CHUNK,), jnp.int32),
        pltpu.VMEM((CHUNK,), jnp.int32),
        pltpu.VMEM((CHUNK,), jnp.int32),
        pltpu.VMEM((CHUNK, D), jnp.float32),
        pltpu.VMEM((CHUNK, D), jnp.float32),
        pltpu.VMEM((CHUNK, D), jnp.float32),
        pltpu.VMEM((CHUNK, D), jnp.float32),
        pltpu.VMEM((CHUNK, D), jnp.float32),
        pltpu.VMEM((CHUNK, D), jnp.float32),
        pltpu.SemaphoreType.DMA,
        pltpu.SemaphoreType.DMA,
        pltpu.SemaphoreType.DMA,
        pltpu.SemaphoreType.DMA,
        pltpu.SemaphoreType.DMA,
        pltpu.SemaphoreType.DMA,
        pltpu.SemaphoreType.DMA,
        pltpu.SemaphoreType.DMA,
    ],
)
def _gather_add(ps_hbm, pr_hbm, s_hbm, r_hbm, out_hbm,
                si0, ri0, si1, ri1, sr0, rr0, sr1, rr1, o0, o1,
                ix0, ix1, gs0, gr0, gs1, gr1, st0, st1):
    wid = lax.axis_index("s") * NC + lax.axis_index("c")
    base = wid * EPW

    def issue_idx(c, si, ri, ix):
        off = base + c * CHUNK
        pltpu.async_copy(s_hbm.at[pl.ds(off, CHUNK)], si, ix)
        pltpu.async_copy(r_hbm.at[pl.ds(off, CHUNK)], ri, ix)

    def drain_idx(si, ri, ix):
        pltpu.make_async_copy(s_hbm.at[pl.ds(0, CHUNK)], si, ix).wait()
        pltpu.make_async_copy(r_hbm.at[pl.ds(0, CHUNK)], ri, ix).wait()

    def issue_g(si, ri, sr, rr, gs, gr):
        pltpu.async_copy(ps_hbm.at[si], sr, gs)
        pltpu.async_copy(pr_hbm.at[ri], rr, gr)

    def drain_g(sr, rr, gs, gr):
        pltpu.make_async_copy(ps_hbm.at[si0], sr, gs).wait()
        pltpu.make_async_copy(pr_hbm.at[ri0], rr, gr).wait()

    def drain_st(o, st):
        pltpu.make_async_copy(o, out_hbm.at[pl.ds(0, CHUNK), :], st).wait()

    def add(sr, rr, o):
        def row(i, carry):
            for j in range(D // LANES):
                sl = pl.ds(j * LANES, LANES)
                o[i, sl] = sr[i, sl] + rr[i, sl]
            return carry

        lax.fori_loop(0, CHUNK, row, 0, unroll=4)

    def store(c, o, st):
        pltpu.async_copy(o, out_hbm.at[pl.ds(base + c * CHUNK, CHUNK), :], st)

    issue_idx(0, si0, ri0, ix0)
    drain_idx(si0, ri0, ix0)
    issue_g(si0, ri0, sr0, rr0, gs0, gr0)
    issue_idx(1, si1, ri1, ix1)

    def body(k, carry):
        c0 = 2 * k
        drain_idx(si1, ri1, ix1)
        drain_g(sr0, rr0, gs0, gr0)
        issue_g(si1, ri1, sr1, rr1, gs1, gr1)
        issue_idx(c0 + 2, si0, ri0, ix0)

        @pl.when(k > 0)
        def _():
            drain_st(o0, st0)

        add(sr0, rr0, o0)
        store(c0, o0, st0)

        drain_idx(si0, ri0, ix0)
        drain_g(sr1, rr1, gs1, gr1)
        issue_g(si0, ri0, sr0, rr0, gs0, gr0)
        issue_idx(jnp.minimum(c0 + 3, NCHUNK - 1), si1, ri1, ix1)

        @pl.when(k > 0)
        def _():
            drain_st(o1, st1)

        add(sr1, rr1, o1)
        store(c0 + 1, o1, st1)
        return carry

    lax.fori_loop(0, (NCHUNK - 1) // 2, body, 0)

    drain_g(sr0, rr0, gs0, gr0)
    drain_idx(si1, ri1, ix1)
    drain_st(o0, st0)
    add(sr0, rr0, o0)
    drain_st(o1, st1)
    store(NCHUNK - 1, o0, st0)
    drain_st(o0, st0)


BE = 2560


def _edge_body(g_ref, ea_ref, w1_ref, b1_ref, w2_ref, b2_ref,
               eno_ref, eout_ref, cs_ref):
    i = pl.program_id(0)
    ea = ea_ref[...]
    h = jnp.maximum(
        g_ref[...]
        + jnp.dot(ea, w1_ref[...], preferred_element_type=jnp.float32)
        + b1_ref[...],
        0.0,
    )
    eno = jnp.dot(h, w2_ref[...], preferred_element_type=jnp.float32) + b2_ref[...]
    eno_ref[...] = eno
    eout_ref[...] = eno + ea

    @pl.when(i == 0)
    def _():
        cs_ref[...] = jnp.zeros_like(cs_ref)

    cs_ref[...] += jnp.sum(eno, axis=0, keepdims=True)


def _edge_mlp(g, edge_attr, w1, b1, w2, b2):
    grid = (E // BE,)
    blk = lambda i: (i, 0)
    fix = lambda i: (0, 0)
    return pl.pallas_call(
        _edge_body,
        grid=grid,
        in_specs=[
            pl.BlockSpec((BE, D), blk),
            pl.BlockSpec((BE, D), blk),
            pl.BlockSpec((D, D), fix),
            pl.BlockSpec((1, D), fix),
            pl.BlockSpec((D, D), fix),
            pl.BlockSpec((1, D), fix),
        ],
        out_specs=(
            pl.BlockSpec((BE, D), blk),
            pl.BlockSpec((BE, D), blk),
            pl.BlockSpec((1, D), fix),
        ),
        out_shape=(
            jax.ShapeDtypeStruct((E, D), jnp.float32),
            jax.ShapeDtypeStruct((E, D), jnp.float32),
            jax.ShapeDtypeStruct((1, D), jnp.float32),
        ),
    )(g, edge_attr, w1, b1, w2, b2)


HALF = N // NC
DUST = HALF
AGG_ROWS = HALF + 8 * NS
EPS = E // NS
NCHUNK_S = EPS // CHUNK
ZR = AGG_ROWS // NS
ZR_LAST = AGG_ROWS - ZR * (NS - 1)
ORPS = 312
ORPS_LAST = HALF - ORPS * (NS - 1)


@functools.partial(
    pl.kernel,
    out_type=jax.ShapeDtypeStruct((N, D), jnp.float32),
    mesh=_mesh,
    scratch_types=[
        pltpu.VMEM((CHUNK,), jnp.int32),
        pltpu.VMEM((CHUNK,), jnp.int32),
        pltpu.VMEM((CHUNK, D), jnp.float32),
        pltpu.VMEM((CHUNK, D), jnp.float32),
        pltpu.VMEM_SHARED((AGG_ROWS, D), jnp.float32),
        pltpu.SemaphoreType.DMA,
        pltpu.SemaphoreType.DMA,
        pltpu.SemaphoreType.DMA,
        pltpu.SemaphoreType.DMA,
        pltpu.SemaphoreType.DMA,
        pltpu.SemaphoreType.DMA,
    ],
)
def _scatter_add(eno_hbm, r_hbm, z_hbm, out_hbm,
                 ib0, ib1, rows0, rows1, agg_sh,
                 ix0, ix1, ld0, ld1, sc0, sc1):
    cid = lax.axis_index("c")
    sid = lax.axis_index("s")

    @pl.when(sid < NS - 1)
    def _():
        pltpu.sync_copy(z_hbm.at[pl.ds(sid * ZR, ZR), :],
                        agg_sh.at[pl.ds(sid * ZR, ZR), :])

    @pl.when(sid == NS - 1)
    def _():
        pltpu.sync_copy(z_hbm.at[pl.ds(ZR * (NS - 1), ZR_LAST), :],
                        agg_sh.at[pl.ds(ZR * (NS - 1), ZR_LAST), :])

    plsc.subcore_barrier()

    base = sid * EPS
    lo = cid * HALF
    dust_row = DUST + 8 * sid

    def issue_idx(c, ib, ix):
        pltpu.async_copy(r_hbm.at[pl.ds(base + c * CHUNK, CHUNK)], ib, ix)

    def drain_idx(ib, ix):
        pltpu.make_async_copy(r_hbm.at[pl.ds(0, CHUNK)], ib, ix).wait()

    def clamp(ib):
        for j in range(CHUNK // LANES):
            sl = pl.ds(j * LANES, LANES)
            v = ib[sl] - lo
            ok = (v >= 0) & (v < HALF)
            ib[sl] = jnp.where(ok, v, dust_row)

    def issue_ld(c, rows, ld):
        pltpu.async_copy(eno_hbm.at[pl.ds(base + c * CHUNK, CHUNK), :], rows, ld)

    def drain_ld(rows, ld):
        pltpu.make_async_copy(eno_hbm.at[pl.ds(0, CHUNK), :], rows, ld).wait()

    def issue_sc(rows, ib, sc):
        pltpu.async_copy(rows, agg_sh.at[ib], sc, add=True)

    def drain_sc(rows, sc):
        pltpu.make_async_copy(rows, agg_sh.at[ib0], sc).wait()

    issue_idx(0, ib0, ix0)
    issue_ld(0, rows0, ld0)

    def body(k, carry):
        c0 = 2 * k
        drain_idx(ib0, ix0)
        clamp(ib0)
        drain_ld(rows0, ld0)
        issue_sc(rows0, ib0, sc0)

        @pl.when(k > 0)
        def _():
            drain_sc(rows1, sc1)

        issue_idx(c0 + 1, ib1, ix1)
        issue_ld(c0 + 1, rows1, ld1)
        drain_idx(ib1, ix1)
        clamp(ib1)
        drain_ld(rows1, ld1)
        issue_sc(rows1, ib1, sc1)

        drain_sc(rows0, sc0)
        issue_idx(c0 + 2, ib0, ix0)
        issue_ld(c0 + 2, rows0, ld0)
        return carry

    lax.fori_loop(0, NCHUNK_S // 2 - 1, body, 0)

    drain_idx(ib0, ix0)
    clamp(ib0)
    drain_ld(rows0, ld0)
    issue_sc(rows0, ib0, sc0)
    drain_sc(rows1, sc1)
    issue_idx(NCHUNK_S - 1, ib1, ix1)
    issue_ld(NCHUNK_S - 1, rows1, ld1)
    drain_idx(ib1, ix1)
    clamp(ib1)
    drain_ld(rows1, ld1)
    issue_sc(rows1, ib1, sc1)
    drain_sc(rows0, sc0)
    drain_sc(rows1, sc1)

    plsc.subcore_barrier()

    @pl.when(sid < NS - 1)
    def _():
        pltpu.sync_copy(agg_sh.at[pl.ds(sid * ORPS, ORPS), :],
                        out_hbm.at[pl.ds(cid * HALF + sid * ORPS, ORPS), :])

    @pl.when(sid == NS - 1)
    def _():
        pltpu.sync_copy(
            agg_sh.at[pl.ds(ORPS * (NS - 1), ORPS_LAST), :],
            out_hbm.at[pl.ds(cid * HALF + ORPS * (NS - 1), ORPS_LAST), :])


def _node_body(na_ref, agg_ref, cs_ref, w1a_ref, w1b_ref, b1_ref,
               w2_ref, b2_ref, out_ref):
    na = na_ref[...]
    aggc = agg_ref[...] - cs_ref[...] * (1.0 / N)
    h = jnp.maximum(
        jnp.dot(na, w1a_ref[...], preferred_element_type=jnp.float32)
        + jnp.dot(aggc, w1b_ref[...], preferred_element_type=jnp.float32)
        + b1_ref[...],
        0.0,
    )
    x = jnp.dot(h, w2_ref[...], preferred_element_type=jnp.float32) + b2_ref[...] + na
    out_ref[...] = x - jnp.mean(x, axis=0, keepdims=True)


def _node_mlp(node_attr, agg, cs, w1a, w1b, b1, w2, b2):
    return pl.pallas_call(
        _node_body,
        out_shape=jax.ShapeDtypeStruct((N, D), jnp.float32),
    )(node_attr, agg, cs, w1a, w1b, b1, w2, b2)


def kernel(node_attr, edge_index, edge_attr, W1e, b1e, W2e, b2e,
           W1n, b1n, W2n, b2n):
    senders = edge_index[0]
    receivers = edge_index[1]
    zeros = jnp.zeros((AGG_ROWS, D), jnp.float32)

    w_sr = jnp.concatenate([W1e[:D], W1e[D:2 * D]], axis=1)
    ps, pr = _precompute(node_attr, w_sr)
    g = _gather_add(ps, pr, senders, receivers)
    eno, edge_out, cs = _edge_mlp(
        g, edge_attr, W1e[2 * D:], b1e.reshape(1, D), W2e, b2e.reshape(1, D)
    )
    agg = _scatter_add(eno, receivers, zeros)
    x = _node_mlp(
        node_attr, agg, cs,
        W1n[:D], W1n[D:], b1n.reshape(1, D), W2n, b2n.reshape(1, D),
    )
    return (x, edge_index, edge_out)
